# baseline (speedup 1.0000x reference)
import functools

import jax
import jax.numpy as jnp
from jax import lax
from jax.experimental import pallas as pl
from jax.experimental.pallas import tpu as pltpu
from jax.experimental.pallas import tpu_sc as plsc

BATCH = 16384
EMB = 1000000
DIM = 64

NUM_CORES = 2
NUM_SUBCORES = 16
NUM_WORKERS = NUM_CORES * NUM_SUBCORES
BPW = BATCH // NUM_WORKERS


def _gather_t(nids, u_t, v_t):
    mesh = plsc.VectorSubcoreMesh(core_axis_name="c", subcore_axis_name="s")

    @functools.partial(
        pl.kernel,
        mesh=mesh,
        out_type=jax.ShapeDtypeStruct((2 * DIM, BATCH), jnp.float32),
        scratch_types=[
            pltpu.VMEM((BPW,), jnp.int32),
            pltpu.VMEM((BPW,), jnp.float32),
            pltpu.SemaphoreType.DMA,
        ],
        compiler_params=pltpu.CompilerParams(use_tc_tiling_on_sc=False),
    )
    def k(nids_hbm, u_hbm, v_hbm, out_hbm, idx_v, row_v, sem):
        wid = lax.axis_index("s") * NUM_CORES + lax.axis_index("c")
        base = wid * BPW
        pltpu.async_copy(nids_hbm.at[pl.ds(base, BPW)], idx_v, sem).wait()
        pltpu.async_copy(u_hbm.at[0, pl.ds(base, BPW)], row_v, sem).wait()
        pltpu.sync_copy(row_v, out_hbm.at[0, pl.ds(base, BPW)])
        pltpu.async_copy(v_hbm.at[0, pl.ds(base, BPW)], row_v, sem).wait()
        pltpu.sync_copy(row_v, out_hbm.at[DIM, pl.ds(base, BPW)])

    return k(nids, u_t, v_t)


def kernel(nids, is_start, directed, u_emb, v_emb):
    out_t = _gather_t(nids.astype(jnp.int32), u_emb.T, v_emb.T)
    return out_t.T


# speedup vs baseline: 23.8891x; 23.8891x over previous
import functools

import jax
import jax.numpy as jnp
from jax import lax
from jax.experimental import pallas as pl
from jax.experimental.pallas import tpu as pltpu
from jax.experimental.pallas import tpu_sc as plsc

BATCH = 16384
DIM = 64

NUM_CORES = 2
NUM_SUBCORES = 16
NUM_WORKERS = NUM_CORES * NUM_SUBCORES
BPW = BATCH // NUM_WORKERS
RING = 4
OG = 16
LANES = 16


def _gather_cat_t(nids, u_t, v_t):
    mesh = plsc.VectorSubcoreMesh(core_axis_name="c", subcore_axis_name="s")

    @functools.partial(
        pl.kernel,
        mesh=mesh,
        out_type=jax.ShapeDtypeStruct((BATCH, 2 * DIM), jnp.float32),
        scratch_types=[
            pltpu.VMEM((BPW + OG,), jnp.int32),
            pltpu.VMEM((RING, DIM, 128), jnp.float32),
            pltpu.VMEM((RING, DIM, 128), jnp.float32),
            pltpu.VMEM((2, OG, 2 * DIM), jnp.float32),
            pltpu.SemaphoreType.DMA,
            pltpu.SemaphoreType.DMA,
            pltpu.SemaphoreType.DMA,
            pltpu.SemaphoreType.DMA,
            pltpu.SemaphoreType.DMA,
            pltpu.SemaphoreType.DMA,
            pltpu.SemaphoreType.DMA,
        ],
        compiler_params=pltpu.CompilerParams(needs_layout_passes=False),
    )
    def k(nids_hbm, u_hbm, v_hbm, out_hbm, idx_v, blk_u, blk_v, stage,
          sem_i, r0, r1, r2, r3, o0, o1):
        wid = lax.axis_index("s") * NUM_CORES + lax.axis_index("c")
        base = wid * BPW
        pltpu.async_copy(
            nids_hbm.at[pl.ds(base, BPW)], idx_v.at[pl.ds(0, BPW)], sem_i
        ).wait()

        rsems = (r0, r1, r2, r3)
        osems = (o0, o1)
        rows16 = [lax.broadcasted_iota(jnp.int32, (LANES,), 0) + h * LANES
                  for h in range(DIM // LANES)]

        def fire(s, slot):
            c0 = pl.multiple_of((s >> 7) << 7, 128)
            pltpu.async_copy(u_hbm.at[:, pl.ds(c0, 128)], blk_u.at[slot], rsems[slot])
            pltpu.async_copy(v_hbm.at[:, pl.ds(c0, 128)], blk_v.at[slot], rsems[slot])

        def drain_extract(s, slot, sslot, j):
            pltpu.make_async_copy(u_hbm.at[:, pl.ds(0, 128)], blk_u.at[slot], rsems[slot]).wait()
            pltpu.make_async_copy(v_hbm.at[:, pl.ds(0, 128)], blk_v.at[slot], rsems[slot]).wait()
            cols = jnp.full((LANES,), s & 127, jnp.int32)
            for h in range(DIM // LANES):
                stage[sslot, j, pl.ds(h * LANES, LANES)] = plsc.load_gather(
                    blk_u.at[slot], [rows16[h], cols]
                )
                stage[sslot, j, pl.ds(DIM + h * LANES, LANES)] = plsc.load_gather(
                    blk_v.at[slot], [rows16[h], cols]
                )

        ivec0 = idx_v[pl.ds(0, OG)]
        for p in range(RING):
            fire(ivec0[p], p)

        def body(t, _):
            for ph in range(2):
                gg = t * 2 + ph

                @pl.when(gg >= 2)
                def _():
                    pltpu.make_async_copy(
                        stage.at[ph], out_hbm.at[pl.ds(0, OG)], osems[ph]
                    ).wait()

                ivec = idx_v[pl.ds(gg * OG, OG)]
                nvec = idx_v[pl.ds(gg * OG + RING, OG)]
                for j in range(OG):
                    drain_extract(ivec[j], j % RING, ph, j)

                    @pl.when(gg * OG + j + RING < BPW)
                    def _():
                        fire(nvec[j], j % RING)

                pltpu.async_copy(
                    stage.at[ph], out_hbm.at[pl.ds(base + gg * OG, OG)], osems[ph]
                )
            return ()

        lax.fori_loop(0, BPW // OG // 2, body, ())
        for p in range(2):
            pltpu.make_async_copy(stage.at[p], out_hbm.at[pl.ds(0, OG)], osems[p]).wait()

    return k(nids, u_t, v_t)


def kernel(nids, is_start, directed, u_emb, v_emb):
    return _gather_cat_t(nids.astype(jnp.int32), u_emb.T, v_emb.T)


# speedup vs baseline: 25.0824x; 1.0499x over previous
import functools

import jax
import jax.numpy as jnp
from jax import lax
from jax.experimental import pallas as pl
from jax.experimental.pallas import tpu as pltpu
from jax.experimental.pallas import tpu_sc as plsc

BATCH = 16384
DIM = 64

NUM_CORES = 2
NUM_SUBCORES = 16
NUM_WORKERS = NUM_CORES * NUM_SUBCORES
BPW = BATCH // NUM_WORKERS
AHEAD = 4
SLOTS = 2 * AHEAD
OG = 16
LANES = 16
HALF = DIM // 2


def _gather_cat_t(nids, u_t, v_t):
    mesh = plsc.VectorSubcoreMesh(core_axis_name="c", subcore_axis_name="s")

    @functools.partial(
        pl.kernel,
        mesh=mesh,
        out_type=jax.ShapeDtypeStruct((BATCH, 2 * DIM), jnp.float32),
        scratch_types=[
            pltpu.VMEM((BPW + OG,), jnp.int32),
            pltpu.VMEM((SLOTS, HALF, 128), jnp.float32),
            pltpu.VMEM((SLOTS, HALF, 128), jnp.float32),
            pltpu.VMEM((2, OG, 2 * DIM), jnp.float32),
            pltpu.SemaphoreType.DMA,
            [pltpu.SemaphoreType.DMA] * SLOTS,
            pltpu.SemaphoreType.DMA,
            pltpu.SemaphoreType.DMA,
        ],
        compiler_params=pltpu.CompilerParams(needs_layout_passes=False),
    )
    def k(nids_hbm, u_hbm, v_hbm, out_hbm, idx_v, blk_u, blk_v, stage,
          sem_i, rsems, o0, o1):
        wid = lax.axis_index("s") * NUM_CORES + lax.axis_index("c")
        base = wid * BPW
        pltpu.async_copy(
            nids_hbm.at[pl.ds(base, BPW)], idx_v.at[pl.ds(0, BPW)], sem_i
        ).wait()

        osems = (o0, o1)
        rows16 = [lax.broadcasted_iota(jnp.int32, (LANES,), 0) + (h * LANES) % HALF
                  for h in range(DIM // LANES)]

        def fire(s, a, b):
            c0 = pl.multiple_of((s >> 7) << 7, 128)
            pltpu.async_copy(u_hbm.at[pl.ds(0, HALF), pl.ds(c0, 128)], blk_u.at[a], rsems[a])
            pltpu.async_copy(v_hbm.at[pl.ds(0, HALF), pl.ds(c0, 128)], blk_v.at[a], rsems[a])
            pltpu.async_copy(u_hbm.at[pl.ds(HALF, HALF), pl.ds(c0, 128)], blk_u.at[b], rsems[b])
            pltpu.async_copy(v_hbm.at[pl.ds(HALF, HALF), pl.ds(c0, 128)], blk_v.at[b], rsems[b])

        def drain_extract(s, a, b, sslot, j):
            cols = jnp.full((LANES,), s & 127, jnp.int32)
            pltpu.make_async_copy(u_hbm.at[pl.ds(0, HALF), pl.ds(0, 128)], blk_u.at[a], rsems[a]).wait()
            pltpu.make_async_copy(u_hbm.at[pl.ds(0, HALF), pl.ds(0, 128)], blk_v.at[a], rsems[a]).wait()
            for h in (0, 1):
                stage[sslot, j, pl.ds(h * LANES, LANES)] = plsc.load_gather(
                    blk_u.at[a], [rows16[h], cols]
                )
                stage[sslot, j, pl.ds(DIM + h * LANES, LANES)] = plsc.load_gather(
                    blk_v.at[a], [rows16[h], cols]
                )
            pltpu.make_async_copy(u_hbm.at[pl.ds(0, HALF), pl.ds(0, 128)], blk_u.at[b], rsems[b]).wait()
            pltpu.make_async_copy(u_hbm.at[pl.ds(0, HALF), pl.ds(0, 128)], blk_v.at[b], rsems[b]).wait()
            for h in (2, 3):
                stage[sslot, j, pl.ds(h * LANES, LANES)] = plsc.load_gather(
                    blk_u.at[b], [rows16[h], cols]
                )
                stage[sslot, j, pl.ds(DIM + h * LANES, LANES)] = plsc.load_gather(
                    blk_v.at[b], [rows16[h], cols]
                )

        ivec0 = idx_v[pl.ds(0, OG)]
        for p in range(AHEAD):
            fire(ivec0[p], (2 * p) % SLOTS, (2 * p + 1) % SLOTS)

        def body(t, _):
            for ph in range(2):
                gg = t * 2 + ph

                @pl.when(gg >= 2)
                def _():
                    pltpu.make_async_copy(
                        stage.at[ph], out_hbm.at[pl.ds(0, OG)], osems[ph]
                    ).wait()

                ivec = idx_v[pl.ds(gg * OG, OG)]
                nvec = idx_v[pl.ds(gg * OG + AHEAD, OG)]
                for j in range(OG):
                    a = (2 * j) % SLOTS
                    b = (2 * j + 1) % SLOTS
                    drain_extract(ivec[j], a, b, ph, j)

                    @pl.when(gg * OG + j + AHEAD < BPW)
                    def _():
                        fire(nvec[j], a, b)

                pltpu.async_copy(
                    stage.at[ph], out_hbm.at[pl.ds(base + gg * OG, OG)], osems[ph]
                )
            return ()

        lax.fori_loop(0, BPW // OG // 2, body, ())
        for p in range(2):
            pltpu.make_async_copy(stage.at[p], out_hbm.at[pl.ds(0, OG)], osems[p]).wait()

    return k(nids, u_t, v_t)


def kernel(nids, is_start, directed, u_emb, v_emb):
    return _gather_cat_t(nids.astype(jnp.int32), u_emb.T, v_emb.T)
